# baseline (speedup 1.0000x reference)
import functools

import jax
import jax.numpy as jnp
from jax import lax
from jax.experimental import pallas as pl
from jax.experimental.pallas import tpu as pltpu
from jax.experimental.pallas import tpu_sc as plsc

N, D, G = 50000, 256, 128
NC, NS, L = 2, 16, 16
NW = NC * NS
R = 80
NSC = 24000
SC_CHUNKS = NSC // R
K_MAX = (SC_CHUNKS + NW - 1) // NW
BN = 2000
NTB = (N - NSC) // BN
BNB = 5000
NB = N // BNB

_NSL = D // L


def _process_chunk(xbuf, idxbuf, acc, cnt):
    zeros = tuple(jnp.zeros((L,), jnp.float32) for _ in range(_NSL))
    ones16 = jnp.ones((L,), jnp.float32)

    def flush(seg, cv, a):
        plsc.addupdate(cnt.at[pl.ds(seg * L, L)], cv)
        for c in range(_NSL):
            plsc.addupdate(acc.at[pl.ds(seg * D + c * L, L)], a[c])

    def body(g, carry):
        seg_prev = carry[0]
        cv = carry[1]
        a = carry[2:]
        segv = idxbuf[pl.ds(g * L, L)]
        for r16 in range(L):
            seg = segv[r16]
            r = g * L + r16

            def do_flush(args2):
                sp2, cv2, av2 = args2
                flush(sp2, cv2, av2)
                return (jnp.zeros((L,), jnp.float32),) + zeros

            def keep(args2):
                return (args2[1],) + tuple(args2[2])

            res = lax.cond(seg != seg_prev, do_flush, keep, (seg_prev, cv, a))
            cv = res[0]
            a = tuple(res[1:])
            a = tuple(a[c] + xbuf[r, pl.ds(c * L, L)] for c in range(_NSL))
            cv = cv + ones16
            seg_prev = seg
        return (seg_prev, cv) + a

    first = idxbuf[pl.ds(0, L)]
    init = (first[0], jnp.zeros((L,), jnp.float32)) + zeros
    carry = lax.fori_loop(0, R // L, body, init)
    flush(carry[0], carry[1], carry[2:])


def _segsum_sc_body(x_hbm, batch_hbm, sums_hbm, cnt_hbm,
                    xbuf0, xbuf1, idx0, idx1, acc, cnt,
                    semx0, semx1, semi0, semi1):
    cid = lax.axis_index("c")
    sid = lax.axis_index("s")
    wid = sid * NC + cid
    xbufs, idxs = (xbuf0, xbuf1), (idx0, idx1)
    semxs, semis = (semx0, semx1), (semi0, semi1)

    zero16 = jnp.zeros((L,), jnp.float32)

    def zbody(r, _):
        for c in range(_NSL):
            acc[pl.ds(r * D + c * L, L)] = zero16
        cnt[pl.ds(r * L, L)] = zero16
        return 0

    lax.fori_loop(0, G, zbody, 0)

    def start(k):
        pb = k % 2
        j = wid + NW * k
        base = j * R

        @pl.when(j < SC_CHUNKS)
        def _():
            pltpu.async_copy(batch_hbm.at[pl.ds(base, R)], idxs[pb],
                             semis[pb])
            pltpu.async_copy(x_hbm.at[pl.ds(base, R), :], xbufs[pb],
                             semxs[pb])

    start(0)
    for k in range(K_MAX):
        pb = k % 2
        j = wid + NW * k
        base = j * R
        if k + 1 < K_MAX:
            start(k + 1)

        @pl.when(j < SC_CHUNKS)
        def _full():
            pltpu.make_async_copy(batch_hbm.at[pl.ds(base, R)], idxs[pb],
                                  semis[pb]).wait()
            pltpu.make_async_copy(x_hbm.at[pl.ds(base, R), :], xbufs[pb],
                                  semxs[pb]).wait()
            _process_chunk(xbufs[pb], idxs[pb], acc, cnt)

    pltpu.sync_copy(acc, sums_hbm.at[wid])
    pltpu.sync_copy(cnt, cnt_hbm.at[wid])


def _segsum_sc(x, batch):
    mesh = plsc.VectorSubcoreMesh(core_axis_name="c", subcore_axis_name="s")
    return pl.kernel(
        _segsum_sc_body,
        out_type=[
            jax.ShapeDtypeStruct((NW, G * D), jnp.float32),
            jax.ShapeDtypeStruct((NW, G * L), jnp.float32),
        ],
        mesh=mesh,
        scratch_types=[
            pltpu.VMEM((R, D), jnp.float32),
            pltpu.VMEM((R, D), jnp.float32),
            pltpu.VMEM((R,), jnp.int32),
            pltpu.VMEM((R,), jnp.int32),
            pltpu.VMEM((G * D,), jnp.float32),
            pltpu.VMEM((G * L,), jnp.float32),
            pltpu.SemaphoreType.DMA,
            pltpu.SemaphoreType.DMA,
            pltpu.SemaphoreType.DMA,
            pltpu.SemaphoreType.DMA,
        ],
    )(x, batch)


def _segsum_tc_body(batch_ref, x_ref, sums_ref, counts_ref):
    i = pl.program_id(0)

    @pl.when(i == 0)
    def _init():
        sums_ref[...] = jnp.zeros_like(sums_ref)
        counts_ref[...] = jnp.zeros_like(counts_ref)

    b = batch_ref[0, 0, :]
    ids = lax.broadcasted_iota(jnp.int32, (BN, G), 1)
    onehot = (b[:, None] == ids).astype(jnp.bfloat16)
    sums_ref[...] += lax.dot_general(
        onehot, x_ref[...].astype(jnp.bfloat16), (((0,), (0,)), ((), ())),
        preferred_element_type=jnp.float32)
    counts_ref[...] += jnp.sum(onehot.astype(jnp.float32), axis=0,
                               keepdims=True)


def _ffn_body(sums_sc_ref, cnt_sc_ref, sums_tc_ref, counts_tc_ref,
              W1_ref, b1_ref, W2_ref, b2_ref, gamma_ref, beta_ref, h_ref):
    sums = jnp.sum(sums_sc_ref[...], axis=0) + sums_tc_ref[...]
    counts = (jnp.sum(cnt_sc_ref[...], axis=0)[:, 0]
              + counts_tc_ref[0, :])
    h = sums / jnp.clip(counts, 1.0)[:, None]
    h = jnp.maximum(
        lax.dot_general(h, W1_ref[...], (((1,), (0,)), ((), ())),
                        preferred_element_type=jnp.float32) + b1_ref[0, :],
        0.0)
    h = lax.dot_general(h, W2_ref[...], (((1,), (0,)), ((), ())),
                        preferred_element_type=jnp.float32) + b2_ref[0, :]
    mu = jnp.mean(h, axis=-1, keepdims=True)
    var = jnp.mean((h - mu) ** 2, axis=-1, keepdims=True)
    h = (h - mu) * lax.rsqrt(var + 1e-5) * gamma_ref[0, :] + beta_ref[0, :]
    h_ref[...] = h


def _bcast_body(batch_ref, x_ref, h_ref, out_ref):
    b = batch_ref[0, 0, :]
    ids = lax.broadcasted_iota(jnp.int32, (BNB, G), 1)
    onehot = (b[:, None] == ids).astype(jnp.bfloat16)
    out_ref[...] = x_ref[...] + lax.dot_general(
        onehot, h_ref[...].astype(jnp.bfloat16), (((1,), (0,)), ((), ())),
        preferred_element_type=jnp.float32)


def kernel(x, batch, W1, b1, W2, b2, gamma, beta):
    batch = batch.astype(jnp.int32)
    batch3s = batch.reshape(N // BN, 1, BN)
    batch3b = batch.reshape(NB, 1, BNB)

    sums_sc, cnt_sc = _segsum_sc(x, batch)
    sums_sc = sums_sc.reshape(NW, G, D)
    cnt_sc = cnt_sc.reshape(NW, G, L)

    off = NSC // BN
    sums_tc, counts_tc = pl.pallas_call(
        _segsum_tc_body,
        grid=(NTB,),
        in_specs=[
            pl.BlockSpec((1, 1, BN), lambda i: (off + i, 0, 0)),
            pl.BlockSpec((BN, D), lambda i: (off + i, 0)),
        ],
        out_specs=[
            pl.BlockSpec((G, D), lambda i: (0, 0)),
            pl.BlockSpec((1, G), lambda i: (0, 0)),
        ],
        out_shape=[
            jax.ShapeDtypeStruct((G, D), jnp.float32),
            jax.ShapeDtypeStruct((1, G), jnp.float32),
        ],
    )(batch3s, x)

    h_vn = pl.pallas_call(
        _ffn_body,
        in_specs=[
            pl.BlockSpec((NW, G, D), lambda: (0, 0, 0)),
            pl.BlockSpec((NW, G, L), lambda: (0, 0, 0)),
            pl.BlockSpec((G, D), lambda: (0, 0)),
            pl.BlockSpec((1, G), lambda: (0, 0)),
            pl.BlockSpec(W1.shape, lambda: (0, 0)),
            pl.BlockSpec((1, b1.shape[0]), lambda: (0, 0)),
            pl.BlockSpec(W2.shape, lambda: (0, 0)),
            pl.BlockSpec((1, b2.shape[0]), lambda: (0, 0)),
            pl.BlockSpec((1, D), lambda: (0, 0)),
            pl.BlockSpec((1, D), lambda: (0, 0)),
        ],
        out_specs=pl.BlockSpec((G, D), lambda: (0, 0)),
        out_shape=jax.ShapeDtypeStruct((G, D), jnp.float32),
    )(sums_sc, cnt_sc, sums_tc, counts_tc, W1, b1.reshape(1, -1),
      W2, b2.reshape(1, -1), gamma.reshape(1, -1), beta.reshape(1, -1))

    x_out = pl.pallas_call(
        _bcast_body,
        grid=(NB,),
        in_specs=[
            pl.BlockSpec((1, 1, BNB), lambda i: (i, 0, 0)),
            pl.BlockSpec((BNB, D), lambda i: (i, 0)),
            pl.BlockSpec((G, D), lambda i: (0, 0)),
        ],
        out_specs=pl.BlockSpec((BNB, D), lambda i: (i, 0)),
        out_shape=jax.ShapeDtypeStruct((N, D), jnp.float32),
    )(batch3b, x, h_vn)

    return (x_out, h_vn)


# speedup vs baseline: 1.0577x; 1.0577x over previous
import functools

import jax
import jax.numpy as jnp
from jax import lax
from jax.experimental import pallas as pl
from jax.experimental.pallas import tpu as pltpu
from jax.experimental.pallas import tpu_sc as plsc

N, D, G = 50000, 256, 128
NC, NS, L = 2, 16, 16
NW = NC * NS
R = 128
NSC = 16000
SC_CHUNKS = NSC // R
K_MAX = (SC_CHUNKS + NW - 1) // NW
BN = 2000
NTB = (N - NSC) // BN
BNB = 2000
NB = N // BNB

_NSL = D // L


def _process_chunk(xbuf, idxbuf, acc, cnt):
    zeros = tuple(jnp.zeros((L,), jnp.float32) for _ in range(_NSL))
    ones16 = jnp.ones((L,), jnp.float32)

    def flush(seg, cv, a):
        plsc.addupdate(cnt.at[pl.ds(seg * L, L)], cv)
        for c in range(_NSL):
            plsc.addupdate(acc.at[pl.ds(seg * D + c * L, L)], a[c])

    def body(g, carry):
        seg_prev = carry[0]
        cv = carry[1]
        a = carry[2:]
        segv = idxbuf[pl.ds(g * L, L)]
        for r16 in range(L):
            seg = segv[r16]
            r = g * L + r16

            def do_flush(args2):
                sp2, cv2, av2 = args2
                flush(sp2, cv2, av2)
                return (jnp.zeros((L,), jnp.float32),) + zeros

            def keep(args2):
                return (args2[1],) + tuple(args2[2])

            res = lax.cond(seg != seg_prev, do_flush, keep, (seg_prev, cv, a))
            cv = res[0]
            a = tuple(res[1:])
            a = tuple(a[c] + xbuf[r, pl.ds(c * L, L)] for c in range(_NSL))
            cv = cv + ones16
            seg_prev = seg
        return (seg_prev, cv) + a

    first = idxbuf[pl.ds(0, L)]
    init = (first[0], jnp.zeros((L,), jnp.float32)) + zeros
    carry = lax.fori_loop(0, R // L, body, init)
    flush(carry[0], carry[1], carry[2:])


def _segsum_sc_body(x_hbm, batch_hbm, sums_hbm, cnt_hbm,
                    xbuf0, xbuf1, idx0, idx1, acc, cnt,
                    semx0, semx1, semi0, semi1):
    cid = lax.axis_index("c")
    sid = lax.axis_index("s")
    wid = sid * NC + cid
    xbufs, idxs = (xbuf0, xbuf1), (idx0, idx1)
    semxs, semis = (semx0, semx1), (semi0, semi1)

    zero16 = jnp.zeros((L,), jnp.float32)

    def zbody(r, _):
        for c in range(_NSL):
            acc[pl.ds(r * D + c * L, L)] = zero16
        cnt[pl.ds(r * L, L)] = zero16
        return 0

    lax.fori_loop(0, G, zbody, 0)

    def start(k):
        pb = k % 2
        j = wid + NW * k
        base = j * R

        @pl.when(j < SC_CHUNKS)
        def _():
            pltpu.async_copy(batch_hbm.at[pl.ds(base, R)], idxs[pb],
                             semis[pb])
            pltpu.async_copy(x_hbm.at[pl.ds(base, R), :], xbufs[pb],
                             semxs[pb])

    start(0)
    for k in range(K_MAX):
        pb = k % 2
        j = wid + NW * k
        base = j * R
        if k + 1 < K_MAX:
            start(k + 1)

        @pl.when(j < SC_CHUNKS)
        def _full():
            pltpu.make_async_copy(batch_hbm.at[pl.ds(base, R)], idxs[pb],
                                  semis[pb]).wait()
            pltpu.make_async_copy(x_hbm.at[pl.ds(base, R), :], xbufs[pb],
                                  semxs[pb]).wait()
            _process_chunk(xbufs[pb], idxs[pb], acc, cnt)

    pltpu.sync_copy(acc, sums_hbm.at[wid])
    pltpu.sync_copy(cnt, cnt_hbm.at[wid])


def _segsum_sc(x, batch):
    mesh = plsc.VectorSubcoreMesh(core_axis_name="c", subcore_axis_name="s")
    return pl.kernel(
        _segsum_sc_body,
        out_type=[
            jax.ShapeDtypeStruct((NW, G * D), jnp.float32),
            jax.ShapeDtypeStruct((NW, G * L), jnp.float32),
        ],
        mesh=mesh,
        scratch_types=[
            pltpu.VMEM((R, D), jnp.float32),
            pltpu.VMEM((R, D), jnp.float32),
            pltpu.VMEM((R,), jnp.int32),
            pltpu.VMEM((R,), jnp.int32),
            pltpu.VMEM((G * D,), jnp.float32),
            pltpu.VMEM((G * L,), jnp.float32),
            pltpu.SemaphoreType.DMA,
            pltpu.SemaphoreType.DMA,
            pltpu.SemaphoreType.DMA,
            pltpu.SemaphoreType.DMA,
        ],
    )(x, batch)


def _segsum_tc_body(batch_ref, x_ref, sums_ref, counts_ref):
    i = pl.program_id(0)

    @pl.when(i == 0)
    def _init():
        sums_ref[...] = jnp.zeros_like(sums_ref)
        counts_ref[...] = jnp.zeros_like(counts_ref)

    b = batch_ref[0, 0, :]
    ids = lax.broadcasted_iota(jnp.int32, (BN, G), 1)
    onehot = (b[:, None] == ids).astype(jnp.bfloat16)
    sums_ref[...] += lax.dot_general(
        onehot, x_ref[...].astype(jnp.bfloat16), (((0,), (0,)), ((), ())),
        preferred_element_type=jnp.float32)
    counts_ref[...] += jnp.sum(onehot.astype(jnp.float32), axis=0,
                               keepdims=True)


def _bcast_ffn_body(batch_ref, x_ref, sums_sc_ref, cnt_sc_ref, sums_tc_ref,
                    counts_tc_ref, W1_ref, b1_ref, W2_ref, b2_ref,
                    gamma_ref, beta_ref, out_ref, h_ref):
    i = pl.program_id(0)

    @pl.when(i == 0)
    def _ffn():
        sums = jnp.sum(sums_sc_ref[...], axis=0) + sums_tc_ref[...]
        counts = (jnp.sum(cnt_sc_ref[...], axis=0)[:, 0]
                  + counts_tc_ref[0, :])
        h = sums / jnp.clip(counts, 1.0)[:, None]
        h = jnp.maximum(
            lax.dot_general(h, W1_ref[...], (((1,), (0,)), ((), ())),
                            preferred_element_type=jnp.float32) + b1_ref[0, :],
            0.0)
        h = lax.dot_general(h, W2_ref[...], (((1,), (0,)), ((), ())),
                            preferred_element_type=jnp.float32) + b2_ref[0, :]
        mu = jnp.mean(h, axis=-1, keepdims=True)
        var = jnp.mean((h - mu) ** 2, axis=-1, keepdims=True)
        h = (h - mu) * lax.rsqrt(var + 1e-5) * gamma_ref[0, :] + beta_ref[0, :]
        h_ref[...] = h

    b = batch_ref[0, 0, :]
    ids = lax.broadcasted_iota(jnp.int32, (BNB, G), 1)
    onehot = (b[:, None] == ids).astype(jnp.bfloat16)
    out_ref[...] = x_ref[...] + lax.dot_general(
        onehot, h_ref[...].astype(jnp.bfloat16), (((1,), (0,)), ((), ())),
        preferred_element_type=jnp.float32)


def kernel(x, batch, W1, b1, W2, b2, gamma, beta):
    batch = batch.astype(jnp.int32)
    batch3s = batch.reshape(N // BN, 1, BN)
    batch3b = batch.reshape(NB, 1, BNB)

    sums_sc, cnt_sc = _segsum_sc(x, batch)
    sums_sc = sums_sc.reshape(NW, G, D)
    cnt_sc = cnt_sc.reshape(NW, G, L)

    off = NSC // BN
    sums_tc, counts_tc = pl.pallas_call(
        _segsum_tc_body,
        grid=(NTB,),
        in_specs=[
            pl.BlockSpec((1, 1, BN), lambda i: (off + i, 0, 0)),
            pl.BlockSpec((BN, D), lambda i: (off + i, 0)),
        ],
        out_specs=[
            pl.BlockSpec((G, D), lambda i: (0, 0)),
            pl.BlockSpec((1, G), lambda i: (0, 0)),
        ],
        out_shape=[
            jax.ShapeDtypeStruct((G, D), jnp.float32),
            jax.ShapeDtypeStruct((1, G), jnp.float32),
        ],
    )(batch3s, x)

    x_out, h_vn = pl.pallas_call(
        _bcast_ffn_body,
        grid=(NB,),
        in_specs=[
            pl.BlockSpec((1, 1, BNB), lambda i: (i, 0, 0)),
            pl.BlockSpec((BNB, D), lambda i: (i, 0)),
            pl.BlockSpec((NW, G, D), lambda i: (0, 0, 0)),
            pl.BlockSpec((NW, G, L), lambda i: (0, 0, 0)),
            pl.BlockSpec((G, D), lambda i: (0, 0)),
            pl.BlockSpec((1, G), lambda i: (0, 0)),
            pl.BlockSpec(W1.shape, lambda i: (0, 0)),
            pl.BlockSpec((1, b1.shape[0]), lambda i: (0, 0)),
            pl.BlockSpec(W2.shape, lambda i: (0, 0)),
            pl.BlockSpec((1, b2.shape[0]), lambda i: (0, 0)),
            pl.BlockSpec((1, D), lambda i: (0, 0)),
            pl.BlockSpec((1, D), lambda i: (0, 0)),
        ],
        out_specs=[
            pl.BlockSpec((BNB, D), lambda i: (i, 0)),
            pl.BlockSpec((G, D), lambda i: (0, 0)),
        ],
        out_shape=[
            jax.ShapeDtypeStruct((N, D), jnp.float32),
            jax.ShapeDtypeStruct((G, D), jnp.float32),
        ],
    )(batch3b, x, sums_sc, cnt_sc, sums_tc, counts_tc, W1,
      b1.reshape(1, -1), W2, b2.reshape(1, -1), gamma.reshape(1, -1),
      beta.reshape(1, -1))

    return (x_out, h_vn)
